# baseline (speedup 1.0000x reference)
import functools

import jax
import jax.numpy as jnp
from jax import lax
from jax.experimental import pallas as pl
from jax.experimental.pallas import tpu as pltpu
from jax.experimental.pallas import tpu_sc as plsc


@functools.cache
def _sc_gather(vocab, dpad, batch, chunks):
    info = plsc.get_sparse_core_info()
    nw = info.num_cores * info.num_subcores
    assert batch % (8 * nw) == 0 and dpad % 128 == 0
    b_per_w = batch // nw
    assert b_per_w % chunks == 0
    b_chunk = b_per_w // chunks

    mesh = plsc.VectorSubcoreMesh(core_axis_name="c", subcore_axis_name="s")

    @functools.partial(
        pl.kernel,
        mesh=mesh,
        out_type=jax.ShapeDtypeStruct((batch, dpad), jnp.float32),
        scratch_types=[
            pltpu.VMEM((b_per_w,), jnp.int32),
            pltpu.VMEM((b_chunk, dpad), jnp.float32),
            pltpu.SemaphoreType.DMA,
        ],
    )
    def gather(table_hbm, idx_hbm, out_hbm, idx_v, rows_v, sem):
        wid = lax.axis_index("s") * info.num_cores + lax.axis_index("c")
        base = wid * b_per_w
        pltpu.sync_copy(idx_hbm.at[pl.ds(base, b_per_w)], idx_v)
        for c in range(chunks):
            pltpu.async_copy(
                table_hbm.at[idx_v.at[pl.ds(c * b_chunk, b_chunk)]], rows_v, sem
            ).wait()
            pltpu.sync_copy(rows_v, out_hbm.at[pl.ds(base + c * b_chunk, b_chunk)])

    return gather


def _ln(x, g, b, eps=1e-5):
    m = jnp.mean(x, axis=-1, keepdims=True)
    c = x - m
    v = jnp.mean(c * c, axis=-1, keepdims=True)
    return c * lax.rsqrt(v + eps) * g + b


def _head_body(x_ref, p_ref, w_ref, bias_ref, o_ref):
    d = p_ref.shape[1]
    rows, cols = o_ref.shape[0], o_ref.shape[1]
    x = x_ref[:, :d]
    p = p_ref[...]
    x = _ln(x, p[0:1, :], p[1:2, :])
    x = _ln(x, p[2:3, :], p[3:4, :])
    x = _ln(x, p[4:5, :], p[5:6, :])
    y = jnp.dot(x, w_ref[...], preferred_element_type=jnp.float32) + bias_ref[...]
    slots = x_ref.shape[0] // rows
    for r in range(rows):
        o_ref[r] = y[r * slots : r * slots + cols]


@functools.cache
def _head(rows, cols, slots, dpad, d, vocab_out, block_rows):
    grid = rows // block_rows
    return pl.pallas_call(
        _head_body,
        grid=(grid,),
        in_specs=[
            pl.BlockSpec((block_rows * slots, dpad), lambda i: (i, 0)),
            pl.BlockSpec((6, d), lambda i: (0, 0)),
            pl.BlockSpec((d, vocab_out), lambda i: (0, 0)),
            pl.BlockSpec((1, vocab_out), lambda i: (0, 0)),
        ],
        out_specs=pl.BlockSpec((block_rows, cols, vocab_out), lambda i: (i, 0, 0)),
        out_shape=jax.ShapeDtypeStruct((rows, cols, vocab_out), jnp.float32),
    )


def kernel(indices, emb, g0, b0, g1, b1, gf, bf, W, bias):
    vocab, d = emb.shape
    vocab_out = W.shape[1]
    rows, cols = indices.shape
    slots = 128
    dpad = 128
    idx = jnp.pad(indices.astype(jnp.int32), ((0, 0), (0, slots - cols))).reshape(-1)
    batch = rows * slots
    emb_pad = jnp.pad(emb, ((0, 0), (0, dpad - d)))

    gathered = _sc_gather(vocab, dpad, batch, 8)(emb_pad, idx)
    params = jnp.stack([g0, b0, g1, b1, gf, bf], axis=0)
    out = _head(rows, cols, slots, dpad, d, vocab_out, 8)(
        gathered, params, W, bias.reshape(1, vocab_out)
    )
    return out


# speedup vs baseline: 6.3756x; 6.3756x over previous
import functools

import jax
import jax.numpy as jnp
from jax import lax
from jax.experimental import pallas as pl
from jax.experimental.pallas import tpu as pltpu
from jax.experimental.pallas import tpu_sc as plsc


@functools.cache
def _sc_gather(vocab, dpad, batch, chunks):
    info = plsc.get_sparse_core_info()
    nw = info.num_cores * info.num_subcores
    assert batch % (8 * nw) == 0 and dpad % 128 == 0
    b_per_w = batch // nw
    assert b_per_w % chunks == 0
    b_chunk = b_per_w // chunks

    mesh = plsc.VectorSubcoreMesh(core_axis_name="c", subcore_axis_name="s")

    @functools.partial(
        pl.kernel,
        mesh=mesh,
        out_type=jax.ShapeDtypeStruct((batch, dpad), jnp.float32),
        scratch_types=[
            pltpu.VMEM((b_per_w,), jnp.int32),
            pltpu.VMEM((b_chunk, dpad), jnp.float32),
            pltpu.SemaphoreType.DMA,
        ],
    )
    def gather(table_hbm, idx_hbm, out_hbm, idx_v, rows_v, sem):
        wid = lax.axis_index("s") * info.num_cores + lax.axis_index("c")
        base = wid * b_per_w
        pltpu.sync_copy(idx_hbm.at[pl.ds(base, b_per_w)], idx_v)
        for c in range(chunks):
            pltpu.async_copy(
                table_hbm.at[idx_v.at[pl.ds(c * b_chunk, b_chunk)]], rows_v, sem
            ).wait()
            pltpu.sync_copy(rows_v, out_hbm.at[pl.ds(base + c * b_chunk, b_chunk)])

    return gather


def _ln(x, g, b, eps=1e-5):
    m = jnp.mean(x, axis=-1, keepdims=True)
    c = x - m
    v = jnp.mean(c * c, axis=-1, keepdims=True)
    return c * lax.rsqrt(v + eps) * g + b


def _head_body(x_ref, p_ref, w_ref, bias_ref, o_ref):
    d = p_ref.shape[1]
    rows, cols = o_ref.shape[0], o_ref.shape[1]
    x = x_ref[:, :d]
    p = p_ref[...]
    x = _ln(x, p[0:1, :], p[1:2, :])
    x = _ln(x, p[2:3, :], p[3:4, :])
    x = _ln(x, p[4:5, :], p[5:6, :])
    y = jnp.dot(x, w_ref[...], preferred_element_type=jnp.float32) + bias_ref[...]
    slots = x_ref.shape[0] // rows
    for r in range(rows):
        o_ref[r] = y[r * slots : r * slots + cols]


@functools.cache
def _head(rows, cols, slots, dpad, d, vocab_out, block_rows):
    grid = rows // block_rows
    return pl.pallas_call(
        _head_body,
        grid=(grid,),
        in_specs=[
            pl.BlockSpec((block_rows * slots, dpad), lambda i: (i, 0)),
            pl.BlockSpec((6, d), lambda i: (0, 0)),
            pl.BlockSpec((d, vocab_out), lambda i: (0, 0)),
            pl.BlockSpec((1, vocab_out), lambda i: (0, 0)),
        ],
        out_specs=pl.BlockSpec((block_rows, cols, vocab_out), lambda i: (i, 0, 0)),
        out_shape=jax.ShapeDtypeStruct((rows, cols, vocab_out), jnp.float32),
    )


def kernel(indices, emb, g0, b0, g1, b1, gf, bf, W, bias):
    vocab, d = emb.shape
    vocab_out = W.shape[1]
    rows, cols = indices.shape
    slots = 128
    dpad = 128
    filler = jax.lax.broadcasted_iota(jnp.int32, (rows, slots), 1) % vocab
    valid = jax.lax.broadcasted_iota(jnp.int32, (rows, slots), 1) < cols
    idxp = jnp.where(
        valid,
        jnp.pad(indices.astype(jnp.int32), ((0, 0), (0, slots - cols))),
        filler,
    )
    idx = idxp.reshape(-1)
    batch = rows * slots
    emb_pad = jnp.pad(emb, ((0, 0), (0, dpad - d)))

    gathered = _sc_gather(vocab, dpad, batch, 8)(emb_pad, idx)
    params = jnp.stack([g0, b0, g1, b1, gf, bf], axis=0)
    out = _head(rows, cols, slots, dpad, d, vocab_out, 8)(
        gathered, params, W, bias.reshape(1, vocab_out)
    )
    return out


# speedup vs baseline: 9.5318x; 1.4950x over previous
import functools

import jax
import jax.numpy as jnp
from jax import lax
from jax.experimental import pallas as pl
from jax.experimental.pallas import tpu as pltpu
from jax.experimental.pallas import tpu_sc as plsc


@functools.cache
def _sc_gather(vocab, dpad, rows, cols, chunks):
    info = plsc.get_sparse_core_info()
    nw = info.num_cores * info.num_subcores
    assert rows % nw == 0 and dpad % 128 == 0
    r_per_w = rows // nw
    assert r_per_w % chunks == 0
    r_chunk = r_per_w // chunks

    mesh = plsc.VectorSubcoreMesh(core_axis_name="c", subcore_axis_name="s")

    @functools.partial(
        pl.kernel,
        mesh=mesh,
        out_type=jax.ShapeDtypeStruct((rows, cols, dpad), jnp.float32),
        scratch_types=[
            pltpu.VMEM((r_per_w, cols), jnp.int32),
            pltpu.VMEM((r_chunk, cols, dpad), jnp.float32),
            pltpu.SemaphoreType.DMA,
        ],
    )
    def gather(table_hbm, idx_hbm, out_hbm, idx_v, rows_v, sem):
        wid = lax.axis_index("s") * info.num_cores + lax.axis_index("c")
        base = wid * r_per_w
        pltpu.sync_copy(idx_hbm.at[pl.ds(base, r_per_w), :], idx_v)
        for c in range(chunks):
            copies = [
                pltpu.async_copy(
                    table_hbm.at[idx_v.at[c * r_chunk + j, :]], rows_v.at[j], sem
                )
                for j in range(r_chunk)
            ]
            for cp in copies:
                cp.wait()
            pltpu.sync_copy(rows_v, out_hbm.at[pl.ds(base + c * r_chunk, r_chunk)])

    return gather


def _ln(x, g, b, eps=1e-5):
    m = jnp.mean(x, axis=-1, keepdims=True)
    c = x - m
    v = jnp.mean(c * c, axis=-1, keepdims=True)
    return c * lax.rsqrt(v + eps) * g + b


def _head_body(x_ref, p_ref, w_ref, bias_ref, o_ref):
    d = p_ref.shape[1]
    x = x_ref[:, :, :d]
    p = p_ref[...]
    x = _ln(x, p[0:1, :], p[1:2, :])
    x = _ln(x, p[2:3, :], p[3:4, :])
    x = _ln(x, p[4:5, :], p[5:6, :])
    w = w_ref[...]
    b = bias_ref[...]
    for r in range(o_ref.shape[0]):
        o_ref[r] = jnp.dot(x[r], w, preferred_element_type=jnp.float32) + b


@functools.cache
def _head(rows, cols, dpad, d, vocab_out, block_rows):
    grid = rows // block_rows
    return pl.pallas_call(
        _head_body,
        grid=(grid,),
        in_specs=[
            pl.BlockSpec((block_rows, cols, dpad), lambda i: (i, 0, 0)),
            pl.BlockSpec((6, d), lambda i: (0, 0)),
            pl.BlockSpec((d, vocab_out), lambda i: (0, 0)),
            pl.BlockSpec((1, vocab_out), lambda i: (0, 0)),
        ],
        out_specs=pl.BlockSpec((block_rows, cols, vocab_out), lambda i: (i, 0, 0)),
        out_shape=jax.ShapeDtypeStruct((rows, cols, vocab_out), jnp.float32),
    )


def kernel(indices, emb, g0, b0, g1, b1, gf, bf, W, bias):
    vocab, d = emb.shape
    vocab_out = W.shape[1]
    rows, cols = indices.shape
    dpad = 128
    emb_pad = jnp.pad(emb, ((0, 0), (0, dpad - d)))

    gathered = _sc_gather(vocab, dpad, rows, cols, 2)(emb_pad, indices.astype(jnp.int32))
    params = jnp.stack([g0, b0, g1, b1, gf, bf], axis=0)
    out = _head(rows, cols, dpad, d, vocab_out, 16)(
        gathered, params, W, bias.reshape(1, vocab_out)
    )
    return out


# speedup vs baseline: 10.0912x; 1.0587x over previous
import functools

import jax
import jax.numpy as jnp
from jax import lax
from jax.experimental import pallas as pl
from jax.experimental.pallas import tpu as pltpu
from jax.experimental.pallas import tpu_sc as plsc


@functools.cache
def _sc_gather(vocab, dpad, rows, cols, chunks):
    info = plsc.get_sparse_core_info()
    nw = info.num_cores * info.num_subcores
    assert rows % nw == 0 and dpad % 128 == 0
    r_per_w = rows // nw
    assert r_per_w % chunks == 0
    r_chunk = r_per_w // chunks

    mesh = plsc.VectorSubcoreMesh(core_axis_name="c", subcore_axis_name="s")

    @functools.partial(
        pl.kernel,
        mesh=mesh,
        out_type=jax.ShapeDtypeStruct((rows, cols, dpad), jnp.float32),
        scratch_types=[
            pltpu.VMEM((r_per_w, cols), jnp.int32),
            pltpu.VMEM((r_chunk, cols, dpad), jnp.float32),
            pltpu.SemaphoreType.DMA,
        ],
    )
    def gather(table_hbm, idx_hbm, out_hbm, idx_v, rows_v, sem):
        wid = lax.axis_index("s") * info.num_cores + lax.axis_index("c")
        base = wid * r_per_w
        pltpu.sync_copy(idx_hbm.at[pl.ds(base, r_per_w), :], idx_v)
        for c in range(chunks):
            copies = [
                pltpu.async_copy(
                    table_hbm.at[idx_v.at[c * r_chunk + j, :]], rows_v.at[j], sem
                )
                for j in range(r_chunk)
            ]
            for cp in copies:
                cp.wait()
            pltpu.sync_copy(rows_v, out_hbm.at[pl.ds(base + c * r_chunk, r_chunk)])

    return gather


def _ln(x, g, b, eps=1e-5):
    m = jnp.mean(x, axis=-1, keepdims=True)
    c = x - m
    v = jnp.mean(c * c, axis=-1, keepdims=True)
    return c * lax.rsqrt(v + eps) * g + b


def _head_body(x_ref, p_ref, w_ref, bias_ref, o_ref):
    d = p_ref.shape[1]
    x = x_ref[:, :, :d]
    p = p_ref[...]
    x = _ln(x, p[0:1, :], p[1:2, :])
    x = _ln(x, p[2:3, :], p[3:4, :])
    x = _ln(x, p[4:5, :], p[5:6, :])
    w = w_ref[...]
    b = bias_ref[...]
    for r in range(o_ref.shape[0]):
        o_ref[r] = jnp.dot(x[r], w, preferred_element_type=jnp.float32) + b


@functools.cache
def _head(rows, cols, dpad, d, vocab_out, block_rows):
    grid = rows // block_rows
    return pl.pallas_call(
        _head_body,
        grid=(grid,),
        in_specs=[
            pl.BlockSpec((block_rows, cols, dpad), lambda i: (i, 0, 0)),
            pl.BlockSpec((6, d), lambda i: (0, 0)),
            pl.BlockSpec((d, vocab_out), lambda i: (0, 0)),
            pl.BlockSpec((1, vocab_out), lambda i: (0, 0)),
        ],
        out_specs=pl.BlockSpec((block_rows, cols, vocab_out), lambda i: (i, 0, 0)),
        out_shape=jax.ShapeDtypeStruct((rows, cols, vocab_out), jnp.float32),
    )


def kernel(indices, emb, g0, b0, g1, b1, gf, bf, W, bias):
    vocab, d = emb.shape
    vocab_out = W.shape[1]
    rows, cols = indices.shape
    dpad = 128
    emb_pad = jnp.pad(emb, ((0, 0), (0, dpad - d)))

    gathered = _sc_gather(vocab, dpad, rows, cols, 2)(emb_pad, indices.astype(jnp.int32))
    params = jnp.stack([g0, b0, g1, b1, gf, bf], axis=0)
    out = _head(rows, cols, dpad, d, vocab_out, 32)(
        gathered, params, W, bias.reshape(1, vocab_out)
    )
    return out


# speedup vs baseline: 10.2054x; 1.0113x over previous
import functools

import jax
import jax.numpy as jnp
from jax import lax
from jax.experimental import pallas as pl
from jax.experimental.pallas import tpu as pltpu
from jax.experimental.pallas import tpu_sc as plsc


@functools.cache
def _sc_gather(vocab, dpad, rows, cols, chunks):
    info = plsc.get_sparse_core_info()
    nw = info.num_cores * info.num_subcores
    assert rows % nw == 0 and dpad % 128 == 0
    r_per_w = rows // nw
    assert r_per_w % chunks == 0
    r_chunk = r_per_w // chunks

    mesh = plsc.VectorSubcoreMesh(core_axis_name="c", subcore_axis_name="s")

    @functools.partial(
        pl.kernel,
        mesh=mesh,
        out_type=jax.ShapeDtypeStruct((rows, cols, dpad), jnp.float32),
        scratch_types=[
            pltpu.VMEM((r_per_w, cols), jnp.int32),
            pltpu.VMEM((r_chunk, cols, dpad), jnp.float32),
            pltpu.SemaphoreType.DMA,
        ],
    )
    def gather(table_hbm, idx_hbm, out_hbm, idx_v, rows_v, sem):
        wid = lax.axis_index("s") * info.num_cores + lax.axis_index("c")
        base = wid * r_per_w
        pltpu.sync_copy(idx_hbm.at[pl.ds(base, r_per_w), :], idx_v)
        for c in range(chunks):
            copies = [
                pltpu.async_copy(
                    table_hbm.at[idx_v.at[c * r_chunk + j, :]], rows_v.at[j], sem
                )
                for j in range(r_chunk)
            ]
            for cp in copies:
                cp.wait()
            pltpu.sync_copy(rows_v, out_hbm.at[pl.ds(base + c * r_chunk, r_chunk)])

    return gather


def _ln(x, g, b, eps=1e-5):
    m = jnp.mean(x, axis=-1, keepdims=True)
    c = x - m
    v = jnp.mean(c * c, axis=-1, keepdims=True)
    return c * lax.rsqrt(v + eps) * g + b


def _head_body(x_ref, p_ref, w_ref, bias_ref, o_ref):
    d = p_ref.shape[1]
    x = x_ref[:, :, :d]
    p = p_ref[...]
    x = _ln(x, p[0:1, :], p[1:2, :])
    x = _ln(x, p[2:3, :], p[3:4, :])
    x = _ln(x, p[4:5, :], p[5:6, :])
    w = w_ref[...]
    b = bias_ref[...]
    for r in range(o_ref.shape[0]):
        o_ref[r] = jnp.dot(x[r], w, preferred_element_type=jnp.float32) + b


@functools.cache
def _head(rows, cols, dpad, d, vocab_out, block_rows):
    grid = rows // block_rows
    return pl.pallas_call(
        _head_body,
        grid=(grid,),
        in_specs=[
            pl.BlockSpec((block_rows, cols, dpad), lambda i: (i, 0, 0)),
            pl.BlockSpec((6, d), lambda i: (0, 0)),
            pl.BlockSpec((d, vocab_out), lambda i: (0, 0)),
            pl.BlockSpec((1, vocab_out), lambda i: (0, 0)),
        ],
        out_specs=pl.BlockSpec((block_rows, cols, vocab_out), lambda i: (i, 0, 0)),
        out_shape=jax.ShapeDtypeStruct((rows, cols, vocab_out), jnp.float32),
    )


def kernel(indices, emb, g0, b0, g1, b1, gf, bf, W, bias):
    vocab, d = emb.shape
    vocab_out = W.shape[1]
    rows, cols = indices.shape
    dpad = 128
    emb_pad = jnp.pad(emb, ((0, 0), (0, dpad - d)))

    gathered = _sc_gather(vocab, dpad, rows, cols, 2)(emb_pad, indices.astype(jnp.int32))
    params = jnp.stack([g0, b0, g1, b1, gf, bf], axis=0)
    out = _head(rows, cols, dpad, d, vocab_out, 64)(
        gathered, params, W, bias.reshape(1, vocab_out)
    )
    return out
